# baseline (speedup 1.0000x reference)
import functools

import jax
import jax.numpy as jnp
from jax import lax
from jax.experimental import pallas as pl
from jax.experimental.pallas import tpu as pltpu
from jax.experimental.pallas import tpu_sc as plsc

N = 10000
D = 128
H = 16
OUT = 10
G = 64
E = 320000

NPAD = 10240
NW = 32
EPW = 10240
ECH = EPW // 128
ECH0 = 80
ECH1 = 80
RPS = NPAD // 16
NB = 16


def _mm_body(x_ref, w_ref, o_ref):
    o_ref[pl.ds(0, N)] = jnp.dot(x_ref[...], w_ref[...], preferred_element_type=jnp.float32)


def _relu_mm_body(a_ref, w_ref, o_ref):
    PR = NPAD * H // 128
    v = a_ref[...]
    h1 = jnp.maximum(v[0:PR] + v[PR:2 * PR], 0.0)
    w = w_ref[...]
    wt = jnp.tile(w, (8, 8))
    pi = lax.broadcasted_iota(jnp.int32, (128, 128), 0) // H
    qi = lax.broadcasted_iota(jnp.int32, (128, 128), 1) // H
    w_big = jnp.where(pi == qi, wt, 0.0)
    o_ref[...] = jnp.dot(h1, w_big, preferred_element_type=jnp.float32)


def _pool_body(a_ref, b_ref, w_ref, o_ref):
    PR = NPAD * H // 128
    PN = N // 8
    v = a_ref[...]
    hp = v[0:PR] + v[PR:2 * PR]
    b = b_ref[...]
    giota = lax.broadcasted_iota(jnp.int32, (G, PN), 0)
    sums = jnp.zeros((G, H), jnp.float32)
    cnt = jnp.zeros((G, 1), jnp.float32)
    for i in range(8):
        ohi = (b[i:i + 1] == giota).astype(jnp.float32)
        sums = sums + jnp.dot(ohi, hp[0:PN, H * i:H * (i + 1)],
                              preferred_element_type=jnp.float32,
                              precision=lax.Precision.HIGHEST)
        cnt = cnt + jnp.sum(ohi, axis=1, keepdims=True)
    pooled = sums / jnp.maximum(cnt, 1.0)
    o_ref[...] = jnp.tanh(jnp.dot(pooled, w_ref[...], preferred_element_type=jnp.float32))


def _edge_scatter(h, ep):
    mesh = plsc.VectorSubcoreMesh(core_axis_name="c", subcore_axis_name="s")

    @functools.partial(
        pl.kernel,
        mesh=mesh,
        out_type=jax.ShapeDtypeStruct((2 * NPAD, H), jnp.float32),
        compiler_params=pltpu.CompilerParams(use_tc_tiling_on_sc=False),
        scratch_types=[
            pltpu.VMEM((ECH0, 128), jnp.int32),
            pltpu.VMEM((ECH0, 128), jnp.int32),
        ]
        + [pltpu.VMEM((128, H), jnp.float32) for _ in range(NB)]
        + [pltpu.SemaphoreType.DMA for _ in range(NB)]
        + [pltpu.VMEM_SHARED((NPAD, H), jnp.float32),
           pltpu.VMEM_SHARED((NPAD, H), jnp.float32)],
    )
    def k(h_hbm, ep_hbm, out_hbm, srcb, dstb, *rest):
        bufs = rest[:NB]
        sems = rest[NB:2 * NB]
        accum = rest[2 * NB]
        htab = rest[2 * NB + 1]
        c = lax.axis_index("c")
        s = lax.axis_index("s")
        cnt = jnp.where(c == 0, ECH0, ECH1)

        pltpu.sync_copy(h_hbm.at[pl.ds(s * RPS, RPS)], htab.at[pl.ds(s * RPS, RPS)])

        for i in range(128):
            bufs[0][i] = jnp.zeros((H,), jnp.float32)
        for kk in range(RPS // 128):
            pltpu.sync_copy(bufs[0], accum.at[pl.ds(s * RPS + kk * 128, 128)])
        plsc.subcore_barrier()

        @pl.when(c == 0)
        def _():
            pltpu.sync_copy(ep_hbm.at[0, pl.ds(s * ECH0, ECH0)], srcb)
            pltpu.sync_copy(ep_hbm.at[1, pl.ds(s * ECH0, ECH0)], dstb)

        @pl.when(c == 1)
        def _():
            base = 16 * ECH0 + s * ECH1
            pltpu.sync_copy(ep_hbm.at[0, pl.ds(base, ECH1)], srcb.at[pl.ds(0, ECH1)])
            pltpu.sync_copy(ep_hbm.at[1, pl.ds(base, ECH1)], dstb.at[pl.ds(0, ECH1)])

        for b in range(NB):
            pltpu.async_copy(htab.at[srcb.at[b]], bufs[b], sems[b])

        def outer(o, carry):
            for b in range(NB):
                j = o * NB + b
                pltpu.make_async_copy(htab.at[srcb.at[j]], bufs[b], sems[b]).wait()
                pltpu.sync_copy(bufs[b], accum.at[dstb.at[j]], add=True)
                pltpu.async_copy(htab.at[srcb.at[j + NB]], bufs[b], sems[b])
            return carry

        lax.fori_loop(0, cnt // NB - 1, outer, 0)
        for b in range(NB):
            j = cnt - NB + b
            pltpu.make_async_copy(htab.at[srcb.at[j]], bufs[b], sems[b]).wait()
            pltpu.sync_copy(bufs[b], accum.at[dstb.at[j]], add=True)
        plsc.subcore_barrier()

        pltpu.sync_copy(
            accum.at[pl.ds(s * RPS, RPS)],
            out_hbm.at[pl.ds(c * NPAD + s * RPS, RPS)],
        )

    return k(h, ep)


def kernel(x, edge_index, batch, W1, W2, Wfc):
    ep = jnp.pad(edge_index, ((0, 0), (0, NW * EPW - E)),
                 constant_values=NPAD - 1).reshape(2, NW * ECH, 128)
    bp = batch.reshape(N // 8, 8).T

    PR = NPAD * H // 128
    hx = pl.pallas_call(_mm_body, out_shape=jax.ShapeDtypeStruct((NPAD, H), jnp.float32))(x, W1)
    agg1 = _edge_scatter(hx, ep)
    h1p = pl.pallas_call(_relu_mm_body, out_shape=jax.ShapeDtypeStruct((PR, 128), jnp.float32))(
        agg1.reshape(2 * PR, 128), W2)
    agg2 = _edge_scatter(h1p.reshape(NPAD, H), ep)
    out = pl.pallas_call(_pool_body, out_shape=jax.ShapeDtypeStruct((G, OUT), jnp.float32))(
        agg2.reshape(2 * PR, 128), bp, Wfc)
    return out


# speedup vs baseline: 1.0453x; 1.0453x over previous
import functools

import jax
import jax.numpy as jnp
from jax import lax
from jax.experimental import pallas as pl
from jax.experimental.pallas import tpu as pltpu
from jax.experimental.pallas import tpu_sc as plsc

N = 10000
D = 128
H = 16
OUT = 10
G = 64
E = 320000

NPAD = 10240
NW = 32
EPW = 10240
ECH = EPW // 128
ECH0 = 88
ECH1 = 72
RPS = NPAD // 16
NB = 8


def _mm_body(x_ref, w_ref, o_ref):
    o_ref[pl.ds(0, N)] = jnp.dot(x_ref[...], w_ref[...], preferred_element_type=jnp.float32)


def _relu_mm_body(a_ref, w_ref, o_ref):
    PR = NPAD * H // 128
    v = a_ref[...]
    h1 = jnp.maximum(v[0:PR] + v[PR:2 * PR], 0.0)
    w = w_ref[...]
    wt = jnp.tile(w, (8, 8))
    pi = lax.broadcasted_iota(jnp.int32, (128, 128), 0) // H
    qi = lax.broadcasted_iota(jnp.int32, (128, 128), 1) // H
    w_big = jnp.where(pi == qi, wt, 0.0)
    o_ref[...] = jnp.dot(h1, w_big, preferred_element_type=jnp.float32)


def _pool_body(a_ref, b_ref, w_ref, o_ref):
    PR = NPAD * H // 128
    PN = N // 8
    v = a_ref[...]
    hp = v[0:PR] + v[PR:2 * PR]
    b = b_ref[...]
    giota = lax.broadcasted_iota(jnp.int32, (G, PN), 0)
    sums = jnp.zeros((G, H), jnp.float32)
    cnt = jnp.zeros((G, 1), jnp.float32)
    for i in range(8):
        ohi = (b[i:i + 1] == giota).astype(jnp.float32)
        sums = sums + jnp.dot(ohi, hp[0:PN, H * i:H * (i + 1)],
                              preferred_element_type=jnp.float32,
                              precision=lax.Precision.HIGHEST)
        cnt = cnt + jnp.sum(ohi, axis=1, keepdims=True)
    pooled = sums / jnp.maximum(cnt, 1.0)
    o_ref[...] = jnp.tanh(jnp.dot(pooled, w_ref[...], preferred_element_type=jnp.float32))


def _edge_scatter(h, ep):
    mesh = plsc.VectorSubcoreMesh(core_axis_name="c", subcore_axis_name="s")

    @functools.partial(
        pl.kernel,
        mesh=mesh,
        out_type=jax.ShapeDtypeStruct((2 * NPAD, H), jnp.float32),
        compiler_params=pltpu.CompilerParams(use_tc_tiling_on_sc=False),
        scratch_types=[
            pltpu.VMEM((ECH0, 128), jnp.int32),
            pltpu.VMEM((ECH0, 128), jnp.int32),
        ]
        + [pltpu.VMEM((128, H), jnp.float32) for _ in range(NB)]
        + [pltpu.SemaphoreType.DMA for _ in range(NB)]
        + [pltpu.VMEM_SHARED((NPAD, H), jnp.float32),
           pltpu.VMEM_SHARED((NPAD, H), jnp.float32)],
    )
    def k(h_hbm, ep_hbm, out_hbm, srcb, dstb, *rest):
        bufs = rest[:NB]
        sems = rest[NB:2 * NB]
        accum = rest[2 * NB]
        htab = rest[2 * NB + 1]
        c = lax.axis_index("c")
        s = lax.axis_index("s")
        cnt = jnp.where(c == 0, ECH0, ECH1)

        pltpu.sync_copy(h_hbm.at[pl.ds(s * RPS, RPS)], htab.at[pl.ds(s * RPS, RPS)])

        for i in range(128):
            bufs[0][i] = jnp.zeros((H,), jnp.float32)
        for kk in range(RPS // 128):
            pltpu.sync_copy(bufs[0], accum.at[pl.ds(s * RPS + kk * 128, 128)])
        plsc.subcore_barrier()

        @pl.when(c == 0)
        def _():
            pltpu.sync_copy(ep_hbm.at[0, pl.ds(s * ECH0, ECH0)], srcb)
            pltpu.sync_copy(ep_hbm.at[1, pl.ds(s * ECH0, ECH0)], dstb)

        @pl.when(c == 1)
        def _():
            base = 16 * ECH0 + s * ECH1
            pltpu.sync_copy(ep_hbm.at[0, pl.ds(base, ECH1)], srcb.at[pl.ds(0, ECH1)])
            pltpu.sync_copy(ep_hbm.at[1, pl.ds(base, ECH1)], dstb.at[pl.ds(0, ECH1)])

        for b in range(NB):
            pltpu.async_copy(htab.at[srcb.at[b]], bufs[b], sems[b])

        def outer(o, carry):
            for b in range(NB):
                j = o * NB + b
                pltpu.make_async_copy(htab.at[srcb.at[j]], bufs[b], sems[b]).wait()
                pltpu.sync_copy(bufs[b], accum.at[dstb.at[j]], add=True)
                pltpu.async_copy(htab.at[srcb.at[j + NB]], bufs[b], sems[b])
            return carry

        lax.fori_loop(0, cnt // NB - 1, outer, 0)
        for b in range(NB):
            j = cnt - NB + b
            pltpu.make_async_copy(htab.at[srcb.at[j]], bufs[b], sems[b]).wait()
            pltpu.sync_copy(bufs[b], accum.at[dstb.at[j]], add=True)
        plsc.subcore_barrier()

        pltpu.sync_copy(
            accum.at[pl.ds(s * RPS, RPS)],
            out_hbm.at[pl.ds(c * NPAD + s * RPS, RPS)],
        )

    return k(h, ep)


def kernel(x, edge_index, batch, W1, W2, Wfc):
    ep = jnp.pad(edge_index, ((0, 0), (0, NW * EPW - E)),
                 constant_values=NPAD - 1).reshape(2, NW * ECH, 128)
    bp = batch.reshape(N // 8, 8).T

    PR = NPAD * H // 128
    hx = pl.pallas_call(_mm_body, out_shape=jax.ShapeDtypeStruct((NPAD, H), jnp.float32))(x, W1)
    agg1 = _edge_scatter(hx, ep)
    h1p = pl.pallas_call(_relu_mm_body, out_shape=jax.ShapeDtypeStruct((PR, 128), jnp.float32))(
        agg1.reshape(2 * PR, 128), W2)
    agg2 = _edge_scatter(h1p.reshape(NPAD, H), ep)
    out = pl.pallas_call(_pool_body, out_shape=jax.ShapeDtypeStruct((G, OUT), jnp.float32))(
        agg2.reshape(2 * PR, 128), bp, Wfc)
    return out
